# Initial kernel scaffold; baseline (speedup 1.0000x reference)
import jax
import jax.numpy as jnp
from jax.experimental import pallas as pl


def kernel(feat, edge_index, W, b):
    raise NotImplementedError("write your pallas kernel here")


# speedup vs baseline: 5.6971x; 5.6971x over previous
import functools

import jax
import jax.numpy as jnp
from jax import lax
from jax.experimental import pallas as pl
from jax.experimental.pallas import tpu as pltpu
from jax.experimental.pallas import tpu_sc as plsc

N = 10000
E = 320000
D = 128
C = 64

NC = 2
NS = 16
CHUNK = 80
EDGES_PER_TILE = E // (NC * NS)
CHUNKS_PER_TILE = EDGES_PER_TILE // CHUNK
NPAD = 10240
ROWS_PER_TILE = NPAD // NS


def _mm_body(x_ref, w_ref, o_ref):
    o_ref[...] = lax.dot_general(
        x_ref[...], w_ref[...], (((1,), (1,)), ((), ())),
        preferred_element_type=jnp.float32)


def _project(feat, W):
    return pl.pallas_call(
        _mm_body,
        grid=(25,),
        in_specs=[
            pl.BlockSpec((400, D), lambda i: (i, 0)),
            pl.BlockSpec((C, D), lambda i: (0, 0)),
        ],
        out_specs=pl.BlockSpec((400, C), lambda i: (i, 0)),
        out_shape=jax.ShapeDtypeStruct((N, C), jnp.float32),
    )(feat, W)


def _add_body(p_ref, bias_ref, o_ref):
    o_ref[...] = p_ref[0] + p_ref[1] + bias_ref[...]


def _combine(P, bias):
    return pl.pallas_call(
        _add_body,
        grid=(20,),
        in_specs=[
            pl.BlockSpec((NC, 512, C), lambda i: (0, i, 0)),
            pl.BlockSpec((1, C), lambda i: (0, 0)),
        ],
        out_specs=pl.BlockSpec((512, C), lambda i: (i, 0)),
        out_shape=jax.ShapeDtypeStruct((NPAD, C), jnp.float32),
    )(P, bias.reshape(1, C))


_MESH = plsc.VectorSubcoreMesh(core_axis_name="c", subcore_axis_name="s")


@functools.partial(
    pl.kernel,
    out_type=jax.ShapeDtypeStruct((NC, NPAD, C), jnp.float32),
    mesh=_MESH,
    scratch_types=[
        pltpu.VMEM((CHUNK,), jnp.int32),
        pltpu.VMEM((CHUNK,), jnp.int32),
        pltpu.VMEM((CHUNK, C), jnp.float32),
        pltpu.VMEM_SHARED((NPAD, C), jnp.float32),
        pltpu.SemaphoreType.DMA,
    ],
    compiler_params=pltpu.CompilerParams(use_tc_tiling_on_sc=False),
)
def _hop(edges_hbm, x_hbm, zeros_hbm, out_hbm, src_v, dst_v, rows_v, accum, sem):
    cid = lax.axis_index("c")
    sid = lax.axis_index("s")
    r0 = sid * ROWS_PER_TILE
    pltpu.sync_copy(zeros_hbm.at[pl.ds(r0, ROWS_PER_TILE)],
                    accum.at[pl.ds(r0, ROWS_PER_TILE)])
    plsc.subcore_barrier()

    base = (sid * NC + cid) * EDGES_PER_TILE

    def body(j, carry):
        off = base + j * CHUNK
        pltpu.sync_copy(edges_hbm.at[pl.ds(E + off, CHUNK)], src_v)
        pltpu.sync_copy(edges_hbm.at[pl.ds(off, CHUNK)], dst_v)
        pltpu.async_copy(x_hbm.at[src_v], rows_v, sem).wait()
        pltpu.sync_copy(rows_v, accum.at[dst_v], add=True)
        return carry

    lax.fori_loop(0, CHUNKS_PER_TILE, body, 0)
    plsc.subcore_barrier()
    pltpu.sync_copy(accum.at[pl.ds(r0, ROWS_PER_TILE)],
                    out_hbm.at[cid, pl.ds(r0, ROWS_PER_TILE)])


NUM_HOPS = 4


def kernel(feat, edge_index, W, b):
    X = _project(feat, W)
    edge_flat = edge_index.reshape(2 * E)
    zeros = jnp.zeros((NPAD, C), jnp.float32)
    zero_bias = jnp.zeros((C,), jnp.float32)
    Y = X
    for i in range(NUM_HOPS):
        P = _hop(edge_flat, Y, zeros)
        Y = _combine(P, b if i == NUM_HOPS - 1 else zero_bias)
    return Y[:N]


# speedup vs baseline: 15.9166x; 2.7938x over previous
import functools

import jax
import jax.numpy as jnp
from jax import lax
from jax.experimental import pallas as pl
from jax.experimental.pallas import tpu as pltpu
from jax.experimental.pallas import tpu_sc as plsc

N = 10000
E = 320000
D = 128
C = 64

NC = 2
NS = 16
NW = NC * NS
CHUNK = 128
NPAD = 10240
ROWS_PER_TILE = NPAD // NS

GRP = 8
CPT = 80
STEPS = CPT // GRP
ECHUNKS = NW * CPT
EPAD = ECHUNKS * CHUNK
NUM_HOPS = 4


def _mm_body(x_ref, w_ref, o_ref):
    o_ref[...] = lax.dot_general(
        x_ref[...], w_ref[...], (((1,), (1,)), ((), ())),
        preferred_element_type=jnp.float32)


def _project(featp, W):
    return pl.pallas_call(
        _mm_body,
        grid=(20,),
        in_specs=[
            pl.BlockSpec((512, D), lambda i: (i, 0)),
            pl.BlockSpec((C, D), lambda i: (0, 0)),
        ],
        out_specs=pl.BlockSpec((512, C), lambda i: (i, 0)),
        out_shape=jax.ShapeDtypeStruct((NPAD, C), jnp.float32),
    )(featp, W)


def _add_body(p_ref, bias_ref, o_ref):
    o_ref[...] = p_ref[0] + p_ref[1] + bias_ref[...]


def _combine(P, bias):
    return pl.pallas_call(
        _add_body,
        grid=(20,),
        in_specs=[
            pl.BlockSpec((NC, 512, C), lambda i: (0, i, 0)),
            pl.BlockSpec((1, C), lambda i: (0, 0)),
        ],
        out_specs=pl.BlockSpec((512, C), lambda i: (i, 0)),
        out_shape=jax.ShapeDtypeStruct((NPAD, C), jnp.float32),
    )(P, bias.reshape(1, C))


_MESH = plsc.VectorSubcoreMesh(core_axis_name="c", subcore_axis_name="s")


@functools.partial(
    pl.kernel,
    out_type=jax.ShapeDtypeStruct((NC, NPAD, C), jnp.float32),
    mesh=_MESH,
    scratch_types=[
        pltpu.VMEM((CPT, CHUNK), jnp.int32),
        pltpu.VMEM((CPT, CHUNK), jnp.int32),
        pltpu.VMEM((GRP, CHUNK, C), jnp.float32),
        pltpu.VMEM_SHARED((NPAD, C), jnp.float32),
        pltpu.SemaphoreType.DMA,
        pltpu.SemaphoreType.DMA,
        pltpu.SemaphoreType.DMA,
    ],
    compiler_params=pltpu.CompilerParams(use_tc_tiling_on_sc=False),
)
def _hop(src_hbm, dst_hbm, x_hbm, zeros_hbm, out_hbm,
         src_v, dst_v, rows_v, accum, gsem, ssem, isem):
    cid = lax.axis_index("c")
    sid = lax.axis_index("s")
    wid = sid * NC + cid
    r0 = sid * ROWS_PER_TILE

    ci = pltpu.async_copy(src_hbm.at[pl.ds(wid * CPT, CPT)], src_v, isem)
    cj = pltpu.async_copy(dst_hbm.at[pl.ds(wid * CPT, CPT)], dst_v, isem)
    pltpu.sync_copy(zeros_hbm.at[pl.ds(r0, ROWS_PER_TILE)],
                    accum.at[pl.ds(r0, ROWS_PER_TILE)])
    ci.wait()
    cj.wait()
    plsc.subcore_barrier()

    def gather(chunk, slot):
        return pltpu.async_copy(
            x_hbm.at[src_v.at[chunk]], rows_v.at[slot], gsem)

    def scatter(chunk, slot):
        return pltpu.async_copy(
            rows_v.at[slot], accum.at[dst_v.at[chunk]], ssem, add=True)

    def body(t, carry):
        c0 = t * GRP
        gs = [gather(c0 + b, b) for b in range(GRP)]
        ss = []
        for b in range(GRP):
            gs[b].wait()
            ss.append(scatter(c0 + b, b))
        for dsc in ss:
            dsc.wait()
        return carry

    lax.fori_loop(0, STEPS, body, 0)
    plsc.subcore_barrier()
    pltpu.sync_copy(accum.at[pl.ds(r0, ROWS_PER_TILE)],
                    out_hbm.at[cid, pl.ds(r0, ROWS_PER_TILE)])


def kernel(feat, edge_index, W, b):
    featp = jnp.pad(feat, ((0, NPAD - N), (0, 0)))
    X = _project(featp, W)

    npad_e = EPAD - E
    pad_rows = N + (jnp.arange(npad_e, dtype=jnp.int32) % (NPAD - N))
    dst2 = jnp.concatenate([edge_index[0], pad_rows]).reshape(ECHUNKS, CHUNK)
    src2 = jnp.concatenate([edge_index[1], pad_rows]).reshape(ECHUNKS, CHUNK)

    zeros = jnp.zeros((NPAD, C), jnp.float32)
    zero_bias = jnp.zeros((C,), jnp.float32)
    Y = X
    for i in range(NUM_HOPS):
        P = _hop(src2, dst2, Y, zeros)
        Y = _combine(P, b if i == NUM_HOPS - 1 else zero_bias)
    return Y[:N]


# speedup vs baseline: 19.0945x; 1.1997x over previous
import functools

import jax
import jax.numpy as jnp
from jax import lax
from jax.experimental import pallas as pl
from jax.experimental.pallas import tpu as pltpu
from jax.experimental.pallas import tpu_sc as plsc

N = 10000
E = 320000
D = 128
C = 64

NC = 2
NS = 16
NW = NC * NS
CHUNK = 128
NPAD = 10240
ROWS_PER_TILE = NPAD // NS

GRP = 8
CPT = 80
STEPS = CPT // GRP
ECHUNKS = NW * CPT
EPAD = ECHUNKS * CHUNK
NUM_HOPS = 4


def _mm_body(x_ref, w_ref, o_ref):
    o_ref[...] = lax.dot_general(
        x_ref[...], w_ref[...], (((1,), (1,)), ((), ())),
        preferred_element_type=jnp.float32)


def _project(featp, W):
    return pl.pallas_call(
        _mm_body,
        grid=(20,),
        in_specs=[
            pl.BlockSpec((512, D), lambda i: (i, 0)),
            pl.BlockSpec((C, D), lambda i: (0, 0)),
        ],
        out_specs=pl.BlockSpec((512, C), lambda i: (i, 0)),
        out_shape=jax.ShapeDtypeStruct((NPAD, C), jnp.float32),
    )(featp, W)


_MESH = plsc.VectorSubcoreMesh(core_axis_name="c", subcore_axis_name="s")

RPW = NPAD // NW


@functools.partial(
    pl.kernel,
    out_type=jax.ShapeDtypeStruct((NPAD, C), jnp.float32),
    mesh=_MESH,
    scratch_types=[
        pltpu.VMEM((RPW, C), jnp.float32),
        pltpu.VMEM((RPW, C), jnp.float32),
        pltpu.VMEM((C,), jnp.float32),
        pltpu.SemaphoreType.DMA,
    ],
    compiler_params=pltpu.CompilerParams(use_tc_tiling_on_sc=False),
)
def _combine(p_hbm, bias_hbm, y_hbm, va, vb, vbias, sem):
    cid = lax.axis_index("c")
    sid = lax.axis_index("s")
    r0 = (sid * NC + cid) * RPW
    c1 = pltpu.async_copy(p_hbm.at[0, pl.ds(r0, RPW)], va, sem)
    c2 = pltpu.async_copy(p_hbm.at[1, pl.ds(r0, RPW)], vb, sem)
    c3 = pltpu.async_copy(bias_hbm, vbias, sem)
    c1.wait()
    c2.wait()
    c3.wait()
    bv = [vbias[pl.ds(k * 16, 16)] for k in range(C // 16)]

    def row(r, carry):
        for k in range(C // 16):
            cs = pl.ds(k * 16, 16)
            va[r, cs] = va[r, cs] + vb[r, cs] + bv[k]
        return carry

    lax.fori_loop(0, RPW, row, 0)
    pltpu.sync_copy(va, y_hbm.at[pl.ds(r0, RPW)])


@functools.partial(
    pl.kernel,
    out_type=jax.ShapeDtypeStruct((NC, NPAD, C), jnp.float32),
    mesh=_MESH,
    scratch_types=[
        pltpu.VMEM((CPT, CHUNK), jnp.int32),
        pltpu.VMEM((CPT, CHUNK), jnp.int32),
        pltpu.VMEM((GRP, CHUNK, C), jnp.float32),
        pltpu.VMEM_SHARED((NPAD, C), jnp.float32),
        pltpu.SemaphoreType.DMA,
        pltpu.SemaphoreType.DMA,
        pltpu.SemaphoreType.DMA,
    ],
    compiler_params=pltpu.CompilerParams(use_tc_tiling_on_sc=False),
)
def _hop(src_hbm, dst_hbm, x_hbm, zeros_hbm, out_hbm,
         src_v, dst_v, rows_v, accum, gsem, ssem, isem):
    cid = lax.axis_index("c")
    sid = lax.axis_index("s")
    wid = sid * NC + cid
    r0 = sid * ROWS_PER_TILE

    ci = pltpu.async_copy(src_hbm.at[pl.ds(wid * CPT, CPT)], src_v, isem)
    cj = pltpu.async_copy(dst_hbm.at[pl.ds(wid * CPT, CPT)], dst_v, isem)
    pltpu.sync_copy(zeros_hbm.at[pl.ds(r0, ROWS_PER_TILE)],
                    accum.at[pl.ds(r0, ROWS_PER_TILE)])
    ci.wait()
    cj.wait()
    plsc.subcore_barrier()

    def gather(chunk, slot):
        return pltpu.async_copy(
            x_hbm.at[src_v.at[chunk]], rows_v.at[slot], gsem)

    def scatter(chunk, slot):
        return pltpu.async_copy(
            rows_v.at[slot], accum.at[dst_v.at[chunk]], ssem, add=True)

    def body(t, carry):
        c0 = t * GRP
        gs = [gather(c0 + b, b) for b in range(GRP)]
        ss = []
        for b in range(GRP):
            gs[b].wait()
            ss.append(scatter(c0 + b, b))
        for dsc in ss:
            dsc.wait()
        return carry

    lax.fori_loop(0, STEPS, body, 0)
    plsc.subcore_barrier()
    pltpu.sync_copy(accum.at[pl.ds(r0, ROWS_PER_TILE)],
                    out_hbm.at[cid, pl.ds(r0, ROWS_PER_TILE)])


def kernel(feat, edge_index, W, b):
    featp = jnp.pad(feat, ((0, NPAD - N), (0, 0)))
    X = _project(featp, W)

    npad_e = EPAD - E
    pad_rows = N + (jnp.arange(npad_e, dtype=jnp.int32) % (NPAD - N))
    dst2 = jnp.concatenate([edge_index[0], pad_rows]).reshape(ECHUNKS, CHUNK)
    src2 = jnp.concatenate([edge_index[1], pad_rows]).reshape(ECHUNKS, CHUNK)

    zeros = jnp.zeros((NPAD, C), jnp.float32)
    zero_bias = jnp.zeros((C,), jnp.float32)
    Y = X
    for i in range(NUM_HOPS):
        P = _hop(src2, dst2, Y, zeros)
        Y = _combine(P, b if i == NUM_HOPS - 1 else zero_bias)
    return Y[:N]


# speedup vs baseline: 19.4729x; 1.0198x over previous
import functools

import jax
import jax.numpy as jnp
from jax import lax
from jax.experimental import pallas as pl
from jax.experimental.pallas import tpu as pltpu
from jax.experimental.pallas import tpu_sc as plsc

N = 10000
E = 320000
D = 128
C = 64

NC = 2
NS = 16
NW = NC * NS
CHUNK = 128
NPAD = 10240
ROWS_PER_TILE = NPAD // NS

GRP = 8
CPT = 78
STEPS = 9
TAIL = CPT - STEPS * GRP
ECHUNKS = E // CHUNK
NUM_HOPS = 4


def _mm_body(x_ref, w_ref, o_ref):
    o_ref[...] = lax.dot_general(
        x_ref[...], w_ref[...], (((1,), (1,)), ((), ())),
        preferred_element_type=jnp.float32)


def _project(featp, W):
    return pl.pallas_call(
        _mm_body,
        grid=(5,),
        in_specs=[
            pl.BlockSpec((2048, D), lambda i: (i, 0)),
            pl.BlockSpec((C, D), lambda i: (0, 0)),
        ],
        out_specs=pl.BlockSpec((2048, C), lambda i: (i, 0)),
        out_shape=jax.ShapeDtypeStruct((NPAD, C), jnp.float32),
    )(featp, W)


_MESH = plsc.VectorSubcoreMesh(core_axis_name="c", subcore_axis_name="s")

RPW = NPAD // NW


@functools.partial(
    pl.kernel,
    out_type=jax.ShapeDtypeStruct((NPAD, C), jnp.float32),
    mesh=_MESH,
    scratch_types=[
        pltpu.VMEM((RPW, C), jnp.float32),
        pltpu.VMEM((RPW, C), jnp.float32),
        pltpu.VMEM((C,), jnp.float32),
        pltpu.SemaphoreType.DMA,
    ],
    compiler_params=pltpu.CompilerParams(use_tc_tiling_on_sc=False),
)
def _combine(p_hbm, bias_hbm, y_hbm, va, vb, vbias, sem):
    cid = lax.axis_index("c")
    sid = lax.axis_index("s")
    r0 = (sid * NC + cid) * RPW
    c1 = pltpu.async_copy(p_hbm.at[0, pl.ds(r0, RPW)], va, sem)
    c2 = pltpu.async_copy(p_hbm.at[1, pl.ds(r0, RPW)], vb, sem)
    c3 = pltpu.async_copy(bias_hbm, vbias, sem)
    c1.wait()
    c2.wait()
    c3.wait()
    bv = [vbias[pl.ds(k * 16, 16)] for k in range(C // 16)]

    def row(r, carry):
        for k in range(C // 16):
            cs = pl.ds(k * 16, 16)
            va[r, cs] = va[r, cs] + vb[r, cs] + bv[k]
        return carry

    lax.fori_loop(0, RPW, row, 0)
    pltpu.sync_copy(va, y_hbm.at[pl.ds(r0, RPW)])


@functools.partial(
    pl.kernel,
    out_type=jax.ShapeDtypeStruct((NC, NPAD, C), jnp.float32),
    mesh=_MESH,
    scratch_types=[
        pltpu.VMEM((CPT + 1, CHUNK), jnp.int32),
        pltpu.VMEM((CPT + 1, CHUNK), jnp.int32),
        pltpu.VMEM((GRP, CHUNK, C), jnp.float32),
        pltpu.VMEM_SHARED((NPAD, C), jnp.float32),
        pltpu.SemaphoreType.DMA,
        pltpu.SemaphoreType.DMA,
        pltpu.SemaphoreType.DMA,
    ],
    compiler_params=pltpu.CompilerParams(use_tc_tiling_on_sc=False),
)
def _hop(src_hbm, dst_hbm, x_hbm, zeros_hbm, out_hbm,
         src_v, dst_v, rows_v, accum, gsem, ssem, isem):
    cid = lax.axis_index("c")
    sid = lax.axis_index("s")
    wid = sid * NC + cid
    r0 = sid * ROWS_PER_TILE

    base = CPT * wid + jnp.minimum(wid, 4)
    ci = pltpu.async_copy(src_hbm.at[pl.ds(base, CPT + 1)], src_v, isem)
    cj = pltpu.async_copy(dst_hbm.at[pl.ds(base, CPT + 1)], dst_v, isem)
    pltpu.sync_copy(zeros_hbm.at[pl.ds(r0, ROWS_PER_TILE)],
                    accum.at[pl.ds(r0, ROWS_PER_TILE)])
    ci.wait()
    cj.wait()
    plsc.subcore_barrier()

    def gather(chunk, slot):
        return pltpu.async_copy(
            x_hbm.at[src_v.at[chunk]], rows_v.at[slot], gsem)

    def scatter(chunk, slot):
        return pltpu.async_copy(
            rows_v.at[slot], accum.at[dst_v.at[chunk]], ssem, add=True)

    def body(t, carry):
        c0 = t * GRP
        gs = [gather(c0 + b, b) for b in range(GRP)]
        ss = []
        for b in range(GRP):
            gs[b].wait()
            ss.append(scatter(c0 + b, b))
        for dsc in ss:
            dsc.wait()
        return carry

    lax.fori_loop(0, STEPS, body, 0)

    gs = [gather(STEPS * GRP + b, b) for b in range(TAIL)]
    ss = []
    for b in range(TAIL):
        gs[b].wait()
        ss.append(scatter(STEPS * GRP + b, b))
    for dsc in ss:
        dsc.wait()

    @pl.when(wid < 4)
    def _():
        gather(CPT, TAIL).wait()
        scatter(CPT, TAIL).wait()

    plsc.subcore_barrier()
    pltpu.sync_copy(accum.at[pl.ds(r0, ROWS_PER_TILE)],
                    out_hbm.at[cid, pl.ds(r0, ROWS_PER_TILE)])


def kernel(feat, edge_index, W, b):
    featp = jnp.pad(feat, ((0, NPAD - N), (0, 0)))
    X = _project(featp, W)

    dst2 = jnp.pad(edge_index[0], (0, CHUNK)).reshape(ECHUNKS + 1, CHUNK)
    src2 = jnp.pad(edge_index[1], (0, CHUNK)).reshape(ECHUNKS + 1, CHUNK)

    zeros = jnp.zeros((NPAD, C), jnp.float32)
    zero_bias = jnp.zeros((C,), jnp.float32)
    Y = X
    for i in range(NUM_HOPS):
        P = _hop(src2, dst2, Y, zeros)
        Y = _combine(P, b if i == NUM_HOPS - 1 else zero_bias)
    return Y[:N]
